# Initial kernel scaffold; baseline (speedup 1.0000x reference)
import jax
import jax.numpy as jnp
from jax.experimental import pallas as pl


def kernel(position_sequence, senders, receivers, enc_node_W1, enc_node_b1, enc_node_W2, enc_node_b2, enc_edge_W1, enc_edge_b1, enc_edge_W2, enc_edge_b2, proc_edge_W1, proc_edge_b1, proc_edge_W2, proc_edge_b2, proc_node_W1, proc_node_b1, proc_node_W2, proc_node_b2, dec_W1, dec_b1, dec_W2, dec_b2):
    raise NotImplementedError("write your pallas kernel here")


# speedup vs baseline: 2.3079x; 2.3079x over previous
import functools

import jax
import jax.numpy as jnp
from jax import lax
from jax.experimental import pallas as pl
from jax.experimental.pallas import tpu as pltpu
from jax.experimental.pallas import tpu_sc as plsc

NC = 2
NS = 16
NW = NC * NS

CH = 80
BLKE = 512
BLKN = 400
NPAD = 10240


def _ln(x):
    m = jnp.mean(x, axis=-1, keepdims=True)
    d = x - m
    v = jnp.mean(d * d, axis=-1, keepdims=True)
    return d * lax.rsqrt(v + 1e-6)


def _relu(x):
    return jnp.maximum(x, 0.0)


def _dot(a, b):
    return jnp.dot(a, b, preferred_element_type=jnp.float32)


def _node_encode_body(nf, W1, b1, W2, b2, Ws, Wr, v_o, ps_o, pr_o):
    h = _relu(_dot(nf[...], W1[...]) + b1[...])
    v = _ln(_dot(h, W2[...]) + b2[...])
    v_o[...] = v
    ps_o[...] = _dot(v, Ws[...])
    pr_o[...] = _dot(v, Wr[...])


def _edge_enc_body(slp, rlp, eW1, eb1, eW2, eb2, e_o):
    d = slp[...] - rlp[...]
    dist = jnp.sqrt(jnp.sum(d * d, axis=-1, keepdims=True))
    lane = lax.broadcasted_iota(jnp.int32, d.shape, 1)
    feat = d + jnp.where(lane == 3, dist, 0.0)
    h = _relu(_dot(feat, eW1[...]) + eb1[...])
    e_o[...] = _ln(_dot(h, eW2[...]) + eb2[...])


def _edge_upd_body(e, gs, gr, W1e, b1, W2, b2, e_o):
    x = e[...]
    h = _relu(_dot(x, W1e[...]) + gs[...] + gr[...] + b1[...])
    e_o[...] = x + _ln(_dot(h, W2[...]) + b2[...])


def _node_upd_proj_body(v, p0, p1, W1v, W1a, b1, W2, b2, Ws, Wr,
                        v_o, ps_o, pr_o):
    x = v[...]
    agg = p0[...] + p1[...]
    h = _relu(_dot(x, W1v[...]) + _dot(agg, W1a[...]) + b1[...])
    vn = x + _ln(_dot(h, W2[...]) + b2[...])
    v_o[...] = vn
    ps_o[...] = _dot(vn, Ws[...])
    pr_o[...] = _dot(vn, Wr[...])


def _decode_body(v, dW1, db1, dW2, db2, lp, pp, out):
    hd = _relu(_dot(v[...], dW1[...]) + db1[...])
    acc = _dot(hd, dW2[...]) + db2[...]
    out[...] = 2.0 * lp[...] - pp[...] + acc


def _full(shape):
    return pl.BlockSpec(shape, lambda i: (0,) * len(shape))


def _rows(blk, width):
    return pl.BlockSpec((blk, width), lambda i: (i, 0))


def _rows_off(blk, width, off_blocks):
    return pl.BlockSpec((blk, width), lambda i: (i + off_blocks, 0))


def _tc_call(body, grid, in_specs, out_specs, out_shape):
    return pl.pallas_call(
        body,
        grid=(grid,),
        in_specs=in_specs,
        out_specs=out_specs,
        out_shape=out_shape,
    )


def _sc_mesh():
    return plsc.VectorSubcoreMesh(core_axis_name="c", subcore_axis_name="s")


@functools.lru_cache(maxsize=None)
def _make_gather(n, e):
    epw = e // NW
    nch = epw // CH
    f32 = jnp.float32

    @functools.partial(
        pl.kernel,
        mesh=_sc_mesh(),
        out_type=[
            jax.ShapeDtypeStruct((e, 128), f32),
            jax.ShapeDtypeStruct((e, 128), f32),
        ],
        scratch_types=[
            pltpu.VMEM((nch, CH), jnp.int32),
            pltpu.VMEM((nch, CH), jnp.int32),
            pltpu.VMEM((CH, 128), f32),
            pltpu.VMEM((CH, 128), f32),
            pltpu.SemaphoreType.DMA,
        ],
    )
    def k(pvs, pvr, snd3, rcv3, gs_o, gr_o, sidx, ridx, bs, br, sem):
        wid = lax.axis_index("s") * NC + lax.axis_index("c")
        base = wid * epw
        pltpu.sync_copy(snd3.at[wid], sidx)
        pltpu.sync_copy(rcv3.at[wid], ridx)

        def body(j, carry):
            off = base + j * CH
            pltpu.async_copy(pvs.at[sidx.at[j]], bs, sem).wait()
            pltpu.sync_copy(bs, gs_o.at[pl.ds(off, CH)])
            pltpu.async_copy(pvr.at[ridx.at[j]], br, sem).wait()
            pltpu.sync_copy(br, gr_o.at[pl.ds(off, CH)])
            return carry

        lax.fori_loop(0, nch, body, 0)

    return k


@functools.lru_cache(maxsize=None)
def _make_scatter(n, e):
    epw = e // NW
    nch = epw // CH
    rpt = NPAD // NS
    rc = 80
    ncopy = rpt // rc
    f32 = jnp.float32

    @functools.partial(
        pl.kernel,
        mesh=_sc_mesh(),
        out_type=[
            jax.ShapeDtypeStruct((NPAD, 128), f32),
            jax.ShapeDtypeStruct((NPAD, 128), f32),
        ],
        scratch_types=[
            pltpu.VMEM((nch, CH), jnp.int32),
            pltpu.VMEM((CH, 128), f32),
            pltpu.VMEM((rc, 128), f32),
            pltpu.VMEM_SHARED((NPAD, 128), f32),
            pltpu.SemaphoreType.DMA,
        ],
    )
    def k(e_hbm, rcv3, out0, out1, idx, rows, zbuf, acc, sem):
        c = lax.axis_index("c")
        s = lax.axis_index("s")
        wid = s * NC + c
        base = wid * epw
        row0 = s * rpt

        def zb(i, carry):
            r = i // 8
            l = (i % 8) * 16
            zbuf[r, pl.ds(l, 16)] = jnp.zeros((16,), f32)
            return carry

        lax.fori_loop(0, rc * 8, zb, 0)
        for kk in range(ncopy):
            pltpu.sync_copy(zbuf, acc.at[pl.ds(row0 + kk * rc, rc)])
        plsc.subcore_barrier()

        pltpu.sync_copy(rcv3.at[wid], idx)

        def body(j, carry):
            pltpu.sync_copy(e_hbm.at[pl.ds(base + j * CH, CH)], rows)
            pltpu.sync_copy(rows, acc.at[idx.at[j]], add=True)
            return carry

        lax.fori_loop(0, nch, body, 0)
        plsc.subcore_barrier()

        for kk in range(ncopy):
            pltpu.sync_copy(acc.at[pl.ds(row0 + kk * rc, rc)], zbuf)

            @pl.when(c == 0)
            def _():
                pltpu.sync_copy(zbuf, out0.at[pl.ds(row0 + kk * rc, rc)])

            @pl.when(c == 1)
            def _():
                pltpu.sync_copy(zbuf, out1.at[pl.ds(row0 + kk * rc, rc)])

    return k


def kernel(position_sequence, senders, receivers,
           enc_node_W1, enc_node_b1, enc_node_W2, enc_node_b2,
           enc_edge_W1, enc_edge_b1, enc_edge_W2, enc_edge_b2,
           proc_edge_W1, proc_edge_b1, proc_edge_W2, proc_edge_b2,
           proc_node_W1, proc_node_b1, proc_node_W2, proc_node_b2,
           dec_W1, dec_b1, dec_W2, dec_b2):
    n, t, d = position_sequence.shape
    e = senders.shape[0]
    s_steps = proc_edge_W1.shape[0]
    L = enc_node_W2.shape[1]
    H = enc_node_W1.shape[1]
    f32 = jnp.float32

    ps = position_sequence.astype(f32)
    vel = (ps[:, 1:] - ps[:, :-1]).reshape(n, (t - 1) * d)
    nf16 = jnp.pad(vel, ((0, 0), (0, 16 - (t - 1) * d)))
    lp = ps[:, -1]
    pp = ps[:, -2]
    lp128 = jnp.pad(lp, ((0, 0), (0, 128 - d)))
    pp128 = jnp.pad(pp, ((0, 0), (0, 128 - d)))

    snd3 = senders.astype(jnp.int32).reshape(NW, (e // NW) // CH, CH)
    rcv3 = receivers.astype(jnp.int32).reshape(NW, (e // NW) // CH, CH)

    nW1p = jnp.pad(enc_node_W1, ((0, 16 - enc_node_W1.shape[0]), (0, 0)))
    eW1p = jnp.pad(enc_edge_W1, ((0, 128 - enc_edge_W1.shape[0]), (0, 0)))
    dW2p = jnp.pad(dec_W2, ((0, 0), (0, 128 - dec_W2.shape[1])))
    db2p = jnp.pad(dec_b2, (0, 128 - dec_b2.shape[0]))

    r1 = lambda b: b.reshape(1, -1)

    peW1e = proc_edge_W1[:, :L]
    peW1s = proc_edge_W1[:, L:2 * L]
    peW1r = proc_edge_W1[:, 2 * L:]
    pnW1v = proc_node_W1[:, :L]
    pnW1a = proc_node_W1[:, L:]
    nxt = list(range(1, s_steps)) + [s_steps - 1]
    Wsn = peW1s[jnp.array(nxt)]
    Wrn = peW1r[jnp.array(nxt)]

    gather = _make_gather(n, e)
    scatter = _make_scatter(n, e)

    ngrid = n // BLKN
    egrid = e // BLKE

    v0, pvs0, pvr0 = _tc_call(
        _node_encode_body, ngrid,
        [_rows(BLKN, 16), _full((16, H)), _full((1, H)), _full((H, L)),
         _full((1, L)), _full((L, L)), _full((L, L))],
        [_rows(BLKN, L)] * 3,
        [jax.ShapeDtypeStruct((n, L), f32)] * 3,
    )(nf16, nW1p, r1(enc_node_b1), enc_node_W2, r1(enc_node_b2),
      peW1s[0], peW1r[0])

    slp, rlp = gather(lp128, lp128, snd3, rcv3)

    e0 = _tc_call(
        _edge_enc_body, egrid,
        [_rows(BLKE, 128), _rows(BLKE, 128),
         _full((128, H)), _full((1, H)), _full((H, L)), _full((1, L))],
        _rows(BLKE, L),
        jax.ShapeDtypeStruct((e, L), f32),
    )(slp, rlp, eW1p, r1(enc_edge_b1), enc_edge_W2, r1(enc_edge_b2))

    def body(carry, ws):
        v, e_lat, pvs, pvr = carry
        (W1e, pb1, pW2, pb2, W1v, W1a, nb1, nW2, nb2, Ws_n, Wr_n) = ws
        gs, gr = gather(pvs, pvr, snd3, rcv3)
        e_lat = _tc_call(
            _edge_upd_body, egrid,
            [_rows(BLKE, L), _rows(BLKE, L), _rows(BLKE, L),
             _full((L, H)), _full((1, H)), _full((H, L)), _full((1, L))],
            _rows(BLKE, L),
            jax.ShapeDtypeStruct((e, L), f32),
        )(e_lat, gs, gr, W1e, pb1, pW2, pb2)
        p0, p1 = scatter(e_lat, rcv3)
        v, pvs, pvr = _tc_call(
            _node_upd_proj_body, ngrid,
            [_rows(BLKN, L), _rows(BLKN, L), _rows(BLKN, L),
             _full((L, H)), _full((L, H)), _full((1, H)),
             _full((H, L)), _full((1, L)), _full((L, L)), _full((L, L))],
            [_rows(BLKN, L)] * 3,
            [jax.ShapeDtypeStruct((n, L), f32)] * 3,
        )(v, p0, p1, W1v, W1a, nb1, nW2, nb2, Ws_n, Wr_n)
        return (v, e_lat, pvs, pvr), 0

    ws_stacked = (peW1e, proc_edge_b1[:, None, :], proc_edge_W2,
                  proc_edge_b2[:, None, :], pnW1v, pnW1a,
                  proc_node_b1[:, None, :], proc_node_W2,
                  proc_node_b2[:, None, :], Wsn, Wrn)
    (v3, _, _, _), _ = lax.scan(body, (v0, e0, pvs0, pvr0), ws_stacked)

    out128 = _tc_call(
        _decode_body, ngrid,
        [_rows(BLKN, L), _full((L, H)), _full((1, H)),
         _full((H, 128)), _full((1, 128)),
         _rows(BLKN, 128), _rows(BLKN, 128)],
        _rows(BLKN, 128),
        jax.ShapeDtypeStruct((n, 128), f32),
    )(v3, dec_W1, r1(dec_b1), dW2p, r1(db2p), lp128, pp128)

    return out128[:, :d]


# speedup vs baseline: 2.8367x; 1.2291x over previous
import functools

import jax
import jax.numpy as jnp
from jax import lax
from jax.experimental import pallas as pl
from jax.experimental.pallas import tpu as pltpu
from jax.experimental.pallas import tpu_sc as plsc

NC = 2
NS = 16
NW = NC * NS

CH = 80
BLKE = 512
BLKN = 400
NPAD = 10240


def _ln(x):
    m = jnp.mean(x, axis=-1, keepdims=True)
    d = x - m
    v = jnp.mean(d * d, axis=-1, keepdims=True)
    return d * lax.rsqrt(v + 1e-6)


def _relu(x):
    return jnp.maximum(x, 0.0)


def _dot(a, b):
    return jnp.dot(a, b, preferred_element_type=jnp.float32)


def _node_encode_body(nf, W1, b1, W2, b2, Ws, Wr, v_o, ps_o, pr_o):
    h = _relu(_dot(nf[...], W1[...]) + b1[...])
    v = _ln(_dot(h, W2[...]) + b2[...])
    v_o[...] = v
    ps_o[...] = _dot(v, Ws[...])
    pr_o[...] = _dot(v, Wr[...])


def _edge_enc_body(slp, rlp, eW1, eb1, eW2, eb2, e_o):
    d = slp[...] - rlp[...]
    dist = jnp.sqrt(jnp.sum(d * d, axis=-1, keepdims=True))
    lane = lax.broadcasted_iota(jnp.int32, d.shape, 1)
    feat = d + jnp.where(lane == 3, dist, 0.0)
    h = _relu(_dot(feat, eW1[...]) + eb1[...])
    e_o[...] = _ln(_dot(h, eW2[...]) + eb2[...])


def _edge_upd_body(e, gs, gr, W1e, b1, W2, b2, e_o):
    x = e[...]
    h = _relu(_dot(x, W1e[...]) + gs[...] + gr[...] + b1[...])
    e_o[...] = x + _ln(_dot(h, W2[...]) + b2[...])


def _node_upd_proj_body(v, p0, p1, W1v, W1a, b1, W2, b2, Ws, Wr,
                        v_o, ps_o, pr_o):
    x = v[...]
    agg = p0[...] + p1[...]
    h = _relu(_dot(x, W1v[...]) + _dot(agg, W1a[...]) + b1[...])
    vn = x + _ln(_dot(h, W2[...]) + b2[...])
    v_o[...] = vn
    ps_o[...] = _dot(vn, Ws[...])
    pr_o[...] = _dot(vn, Wr[...])


def _decode_body(v, dW1, db1, dW2, db2, lp, pp, out):
    hd = _relu(_dot(v[...], dW1[...]) + db1[...])
    acc = _dot(hd, dW2[...]) + db2[...]
    out[...] = 2.0 * lp[...] - pp[...] + acc


def _full(shape):
    return pl.BlockSpec(shape, lambda i: (0,) * len(shape))


def _rows(blk, width):
    return pl.BlockSpec((blk, width), lambda i: (i, 0))


def _rows_off(blk, width, off_blocks):
    return pl.BlockSpec((blk, width), lambda i: (i + off_blocks, 0))


def _tc_call(body, grid, in_specs, out_specs, out_shape):
    return pl.pallas_call(
        body,
        grid=(grid,),
        in_specs=in_specs,
        out_specs=out_specs,
        out_shape=out_shape,
    )


def _sc_mesh():
    return plsc.VectorSubcoreMesh(core_axis_name="c", subcore_axis_name="s")


@functools.lru_cache(maxsize=None)
def _make_gather(n, e):
    epw = e // NW
    nch = epw // CH
    f32 = jnp.float32

    @functools.partial(
        pl.kernel,
        mesh=_sc_mesh(),
        out_type=[
            jax.ShapeDtypeStruct((e, 128), f32),
            jax.ShapeDtypeStruct((e, 128), f32),
        ],
        scratch_types=[
            pltpu.VMEM((nch, CH), jnp.int32),
            pltpu.VMEM((nch, CH), jnp.int32),
        ] + [pltpu.VMEM((CH, 128), f32)] * 8
          + [pltpu.SemaphoreType.DMA] * 8,
    )
    def k(pvs, pvr, snd3, rcv3, gs_o, gr_o, sidx, ridx,
          bs0, bs1, bs2, bs3, br0, br1, br2, br3,
          ss0, ss1, ss2, ss3, sr0, sr1, sr2, sr3):
        wid = lax.axis_index("s") * NC + lax.axis_index("c")
        base = wid * epw
        pltpu.sync_copy(snd3.at[wid], sidx)
        pltpu.sync_copy(rcv3.at[wid], ridx)

        bufs = [(bs0, br0, ss0, sr0), (bs1, br1, ss1, sr1),
                (bs2, br2, ss2, sr2), (bs3, br3, ss3, sr3)]

        def gather_into(j, B):
            bs, br, ss, sr = B
            pltpu.async_copy(pvs.at[sidx.at[j]], bs, ss)
            pltpu.async_copy(pvr.at[ridx.at[j]], br, sr)

        def wait_gather(B):
            bs, br, ss, sr = B
            pltpu.make_async_copy(pvs.at[sidx.at[0]], bs, ss).wait()
            pltpu.make_async_copy(pvr.at[ridx.at[0]], br, sr).wait()

        def wb(j, B):
            bs, br, ss, sr = B
            off = base + j * CH
            pltpu.async_copy(bs, gs_o.at[pl.ds(off, CH)], ss)
            pltpu.async_copy(br, gr_o.at[pl.ds(off, CH)], sr)

        def wait_wb(B):
            bs, br, ss, sr = B
            pltpu.make_async_copy(bs, gs_o.at[pl.ds(base, CH)], ss).wait()
            pltpu.make_async_copy(br, gr_o.at[pl.ds(base, CH)], sr).wait()

        gather_into(0, bufs[0])
        gather_into(1, bufs[1])

        def body(j, carry):
            for m in range(4):
                @pl.when(j % 4 == m)
                def _(m=m):
                    B = bufs[m]
                    C = bufs[(m + 2) % 4]
                    wait_gather(B)
                    wb(j, B)

                    @pl.when(j >= 2)
                    def _():
                        wait_wb(C)

                    @pl.when(j + 2 < nch)
                    def _():
                        gather_into(j + 2, C)
            return carry

        lax.fori_loop(0, nch, body, 0)
        wait_wb(bufs[(nch - 2) % 4])
        wait_wb(bufs[(nch - 1) % 4])

    return k


@functools.lru_cache(maxsize=None)
def _make_scatter(n, e):
    epw = e // NW
    nch = epw // CH
    rpt = NPAD // NS
    rc = 80
    ncopy = rpt // rc
    f32 = jnp.float32

    @functools.partial(
        pl.kernel,
        mesh=_sc_mesh(),
        out_type=[
            jax.ShapeDtypeStruct((NPAD, 128), f32),
            jax.ShapeDtypeStruct((NPAD, 128), f32),
        ],
        scratch_types=[
            pltpu.VMEM((nch, CH), jnp.int32),
            pltpu.VMEM((CH, 128), f32),
            pltpu.VMEM((CH, 128), f32),
            pltpu.VMEM((rc, 128), f32),
            pltpu.VMEM_SHARED((NPAD, 128), f32),
            pltpu.SemaphoreType.DMA,
            pltpu.SemaphoreType.DMA,
        ],
    )
    def k(e_hbm, rcv3, out0, out1, idx, rows, rows2, zbuf, acc, rsem, rsem2):
        c = lax.axis_index("c")
        s = lax.axis_index("s")
        wid = s * NC + c
        base = wid * epw
        row0 = s * rpt

        def zb(i, carry):
            r = i // 8
            l = (i % 8) * 16
            zbuf[r, pl.ds(l, 16)] = jnp.zeros((16,), f32)
            return carry

        lax.fori_loop(0, rc * 8, zb, 0)
        for kk in range(ncopy):
            pltpu.sync_copy(zbuf, acc.at[pl.ds(row0 + kk * rc, rc)])
        plsc.subcore_barrier()

        pltpu.sync_copy(rcv3.at[wid], idx)

        rbufs = [(rows, rsem), (rows2, rsem2)]
        pltpu.async_copy(e_hbm.at[pl.ds(base, CH)], rows, rsem)

        def body(j, carry):
            for m in range(2):
                @pl.when(j % 2 == m)
                def _(m=m):
                    rb, rs = rbufs[m]
                    nb, ns = rbufs[1 - m]

                    @pl.when(j + 1 < nch)
                    def _():
                        pltpu.async_copy(
                            e_hbm.at[pl.ds(base + (j + 1) * CH, CH)], nb, ns)

                    pltpu.make_async_copy(
                        e_hbm.at[pl.ds(base, CH)], rb, rs).wait()
                    pltpu.sync_copy(rb, acc.at[idx.at[j]], add=True)
            return carry

        lax.fori_loop(0, nch, body, 0)
        plsc.subcore_barrier()

        for kk in range(ncopy):
            pltpu.sync_copy(acc.at[pl.ds(row0 + kk * rc, rc)], zbuf)

            @pl.when(c == 0)
            def _():
                pltpu.sync_copy(zbuf, out0.at[pl.ds(row0 + kk * rc, rc)])

            @pl.when(c == 1)
            def _():
                pltpu.sync_copy(zbuf, out1.at[pl.ds(row0 + kk * rc, rc)])

    return k


def kernel(position_sequence, senders, receivers,
           enc_node_W1, enc_node_b1, enc_node_W2, enc_node_b2,
           enc_edge_W1, enc_edge_b1, enc_edge_W2, enc_edge_b2,
           proc_edge_W1, proc_edge_b1, proc_edge_W2, proc_edge_b2,
           proc_node_W1, proc_node_b1, proc_node_W2, proc_node_b2,
           dec_W1, dec_b1, dec_W2, dec_b2):
    n, t, d = position_sequence.shape
    e = senders.shape[0]
    s_steps = proc_edge_W1.shape[0]
    L = enc_node_W2.shape[1]
    H = enc_node_W1.shape[1]
    f32 = jnp.float32

    ps = position_sequence.astype(f32)
    vel = (ps[:, 1:] - ps[:, :-1]).reshape(n, (t - 1) * d)
    nf16 = jnp.pad(vel, ((0, 0), (0, 16 - (t - 1) * d)))
    lp = ps[:, -1]
    pp = ps[:, -2]
    lp128 = jnp.pad(lp, ((0, 0), (0, 128 - d)))
    pp128 = jnp.pad(pp, ((0, 0), (0, 128 - d)))

    snd3 = senders.astype(jnp.int32).reshape(NW, (e // NW) // CH, CH)
    rcv3 = receivers.astype(jnp.int32).reshape(NW, (e // NW) // CH, CH)

    nW1p = jnp.pad(enc_node_W1, ((0, 16 - enc_node_W1.shape[0]), (0, 0)))
    eW1p = jnp.pad(enc_edge_W1, ((0, 128 - enc_edge_W1.shape[0]), (0, 0)))
    dW2p = jnp.pad(dec_W2, ((0, 0), (0, 128 - dec_W2.shape[1])))
    db2p = jnp.pad(dec_b2, (0, 128 - dec_b2.shape[0]))

    r1 = lambda b: b.reshape(1, -1)

    peW1e = proc_edge_W1[:, :L]
    peW1s = proc_edge_W1[:, L:2 * L]
    peW1r = proc_edge_W1[:, 2 * L:]
    pnW1v = proc_node_W1[:, :L]
    pnW1a = proc_node_W1[:, L:]
    nxt = list(range(1, s_steps)) + [s_steps - 1]
    Wsn = peW1s[jnp.array(nxt)]
    Wrn = peW1r[jnp.array(nxt)]

    gather = _make_gather(n, e)
    scatter = _make_scatter(n, e)

    ngrid = n // BLKN
    egrid = e // BLKE

    v0, pvs0, pvr0 = _tc_call(
        _node_encode_body, ngrid,
        [_rows(BLKN, 16), _full((16, H)), _full((1, H)), _full((H, L)),
         _full((1, L)), _full((L, L)), _full((L, L))],
        [_rows(BLKN, L)] * 3,
        [jax.ShapeDtypeStruct((n, L), f32)] * 3,
    )(nf16, nW1p, r1(enc_node_b1), enc_node_W2, r1(enc_node_b2),
      peW1s[0], peW1r[0])

    slp, rlp = gather(lp128, lp128, snd3, rcv3)

    e0 = _tc_call(
        _edge_enc_body, egrid,
        [_rows(BLKE, 128), _rows(BLKE, 128),
         _full((128, H)), _full((1, H)), _full((H, L)), _full((1, L))],
        _rows(BLKE, L),
        jax.ShapeDtypeStruct((e, L), f32),
    )(slp, rlp, eW1p, r1(enc_edge_b1), enc_edge_W2, r1(enc_edge_b2))

    def body(carry, ws):
        v, e_lat, pvs, pvr = carry
        (W1e, pb1, pW2, pb2, W1v, W1a, nb1, nW2, nb2, Ws_n, Wr_n) = ws
        gs, gr = gather(pvs, pvr, snd3, rcv3)
        e_lat = _tc_call(
            _edge_upd_body, egrid,
            [_rows(BLKE, L), _rows(BLKE, L), _rows(BLKE, L),
             _full((L, H)), _full((1, H)), _full((H, L)), _full((1, L))],
            _rows(BLKE, L),
            jax.ShapeDtypeStruct((e, L), f32),
        )(e_lat, gs, gr, W1e, pb1, pW2, pb2)
        p0, p1 = scatter(e_lat, rcv3)
        v, pvs, pvr = _tc_call(
            _node_upd_proj_body, ngrid,
            [_rows(BLKN, L), _rows(BLKN, L), _rows(BLKN, L),
             _full((L, H)), _full((L, H)), _full((1, H)),
             _full((H, L)), _full((1, L)), _full((L, L)), _full((L, L))],
            [_rows(BLKN, L)] * 3,
            [jax.ShapeDtypeStruct((n, L), f32)] * 3,
        )(v, p0, p1, W1v, W1a, nb1, nW2, nb2, Ws_n, Wr_n)
        return (v, e_lat, pvs, pvr), 0

    ws_stacked = (peW1e, proc_edge_b1[:, None, :], proc_edge_W2,
                  proc_edge_b2[:, None, :], pnW1v, pnW1a,
                  proc_node_b1[:, None, :], proc_node_W2,
                  proc_node_b2[:, None, :], Wsn, Wrn)
    (v3, _, _, _), _ = lax.scan(body, (v0, e0, pvs0, pvr0), ws_stacked)

    out128 = _tc_call(
        _decode_body, ngrid,
        [_rows(BLKN, L), _full((L, H)), _full((1, H)),
         _full((H, 128)), _full((1, 128)),
         _rows(BLKN, 128), _rows(BLKN, 128)],
        _rows(BLKN, 128),
        jax.ShapeDtypeStruct((n, 128), f32),
    )(v3, dec_W1, r1(dec_b1), dW2p, r1(db2p), lp128, pp128)

    return out128[:, :d]


# speedup vs baseline: 3.0511x; 1.0756x over previous
import functools

import jax
import jax.numpy as jnp
from jax import lax
from jax.experimental import pallas as pl
from jax.experimental.pallas import tpu as pltpu
from jax.experimental.pallas import tpu_sc as plsc

NC = 2
NS = 16
NW = NC * NS

CH = 80
BLKE = 512
BLKN = 400
NPAD = 10240


def _ln(x):
    m = jnp.mean(x, axis=-1, keepdims=True)
    d = x - m
    v = jnp.mean(d * d, axis=-1, keepdims=True)
    return d * lax.rsqrt(v + 1e-6)


def _relu(x):
    return jnp.maximum(x, 0.0)


def _dot(a, b):
    return jnp.dot(a, b, preferred_element_type=jnp.float32)


def _node_encode_body(nf, W1, b1, W2, b2, Ws, Wr, v_o, ps_o, pr_o):
    h = _relu(_dot(nf[...], W1[...]) + b1[...])
    v = _ln(_dot(h, W2[...]) + b2[...])
    v_o[...] = v
    ps_o[...] = _dot(v, Ws[...])
    pr_o[...] = _dot(v, Wr[...])


def _edge_enc_body(d_in, eW1, eb1, eW2, eb2, e_o):
    d = d_in[...]
    dist = jnp.sqrt(jnp.sum(d * d, axis=-1, keepdims=True))
    lane = lax.broadcasted_iota(jnp.int32, d.shape, 1)
    feat = d + jnp.where(lane == 3, dist, 0.0)
    h = _relu(_dot(feat, eW1[...]) + eb1[...])
    e_o[...] = _ln(_dot(h, eW2[...]) + eb2[...])


def _edge_upd_body(e, g, W1e, b1, W2, b2, e_o):
    x = e[...]
    h = _relu(_dot(x, W1e[...]) + g[...] + b1[...])
    e_o[...] = x + _ln(_dot(h, W2[...]) + b2[...])


def _node_upd_proj_body(v, p0, p1, W1v, W1a, b1, W2, b2, Ws, Wr,
                        v_o, ps_o, pr_o):
    x = v[...]
    agg = p0[...] + p1[...]
    h = _relu(_dot(x, W1v[...]) + _dot(agg, W1a[...]) + b1[...])
    vn = x + _ln(_dot(h, W2[...]) + b2[...])
    v_o[...] = vn
    ps_o[...] = _dot(vn, Ws[...])
    pr_o[...] = _dot(vn, Wr[...])


def _decode_body(v, dW1, db1, dW2, db2, lp, pp, out):
    hd = _relu(_dot(v[...], dW1[...]) + db1[...])
    acc = _dot(hd, dW2[...]) + db2[...]
    out[...] = 2.0 * lp[...] - pp[...] + acc


def _full(shape):
    return pl.BlockSpec(shape, lambda i: (0,) * len(shape))


def _rows(blk, width):
    return pl.BlockSpec((blk, width), lambda i: (i, 0))


def _rows_off(blk, width, off_blocks):
    return pl.BlockSpec((blk, width), lambda i: (i + off_blocks, 0))


def _tc_call(body, grid, in_specs, out_specs, out_shape):
    return pl.pallas_call(
        body,
        grid=(grid,),
        in_specs=in_specs,
        out_specs=out_specs,
        out_shape=out_shape,
    )


def _sc_mesh():
    return plsc.VectorSubcoreMesh(core_axis_name="c", subcore_axis_name="s")


@functools.lru_cache(maxsize=None)
def _make_gather(n, e):
    epw = e // NW
    nch = epw // CH
    f32 = jnp.float32

    @functools.partial(
        pl.kernel,
        mesh=_sc_mesh(),
        out_type=jax.ShapeDtypeStruct((e, 128), f32),
        scratch_types=[
            pltpu.VMEM((nch, CH), jnp.int32),
            pltpu.VMEM((nch, CH), jnp.int32),
        ] + [pltpu.VMEM((CH, 128), f32)] * 4
          + [pltpu.SemaphoreType.DMA] * 8,
    )
    def k(ta, tb, snd3, rcv3, g_o, sidx, ridx,
          b0, b1, b2, b3, sa0, sa1, sa2, sa3, sb0, sb1, sb2, sb3):
        wid = lax.axis_index("s") * NC + lax.axis_index("c")
        base = wid * epw
        pltpu.sync_copy(snd3.at[wid], sidx)
        pltpu.sync_copy(rcv3.at[wid], ridx)

        bufs = [(b0, sa0, sb0), (b1, sa1, sb1), (b2, sa2, sb2),
                (b3, sa3, sb3)]

        def body(j, carry):
            @pl.when(j >= 2)
            def _():
                c3 = j - 2
                for m in range(4):
                    @pl.when(c3 % 4 == m)
                    def _(m=m):
                        b, sa, sb = bufs[m]
                        pltpu.make_async_copy(tb.at[ridx.at[0]], b, sb).wait()
                        pltpu.async_copy(
                            b, g_o.at[pl.ds(base + c3 * CH, CH)], sa)

            @pl.when(jnp.logical_and(j >= 1, j <= nch))
            def _():
                c2 = j - 1
                for m in range(4):
                    @pl.when(c2 % 4 == m)
                    def _(m=m):
                        b, sa, sb = bufs[m]
                        pltpu.make_async_copy(ta.at[sidx.at[0]], b, sa).wait()
                        pltpu.async_copy(tb.at[ridx.at[c2]], b, sb, add=True)

            @pl.when(j < nch)
            def _():
                for m in range(4):
                    @pl.when(j % 4 == m)
                    def _(m=m):
                        b, sa, sb = bufs[m]

                        @pl.when(j >= 4)
                        def _():
                            pltpu.make_async_copy(
                                b, g_o.at[pl.ds(base, CH)], sa).wait()

                        pltpu.async_copy(ta.at[sidx.at[j]], b, sa)
            return carry

        lax.fori_loop(0, nch + 2, body, 0)
        for m in range(4):
            b, sa, sb = bufs[m]
            pltpu.make_async_copy(b, g_o.at[pl.ds(base, CH)], sa).wait()

    return k


@functools.lru_cache(maxsize=None)
def _make_scatter(n, e):
    epw = e // NW
    nch = epw // CH
    rpt = NPAD // NS
    rc = 80
    ncopy = rpt // rc
    f32 = jnp.float32

    @functools.partial(
        pl.kernel,
        mesh=_sc_mesh(),
        out_type=[
            jax.ShapeDtypeStruct((NPAD, 128), f32),
            jax.ShapeDtypeStruct((NPAD, 128), f32),
        ],
        scratch_types=[
            pltpu.VMEM((nch, CH), jnp.int32),
            pltpu.VMEM((CH, 128), f32),
            pltpu.VMEM((CH, 128), f32),
            pltpu.VMEM((rc, 128), f32),
            pltpu.VMEM_SHARED((NPAD, 128), f32),
            pltpu.SemaphoreType.DMA,
            pltpu.SemaphoreType.DMA,
        ],
    )
    def k(e_hbm, rcv3, out0, out1, idx, rows, rows2, zbuf, acc, rsem, rsem2):
        c = lax.axis_index("c")
        s = lax.axis_index("s")
        wid = s * NC + c
        base = wid * epw
        row0 = s * rpt

        def zb(i, carry):
            r = i // 8
            l = (i % 8) * 16
            zbuf[r, pl.ds(l, 16)] = jnp.zeros((16,), f32)
            return carry

        lax.fori_loop(0, rc * 8, zb, 0)
        for kk in range(ncopy):
            pltpu.sync_copy(zbuf, acc.at[pl.ds(row0 + kk * rc, rc)])
        plsc.subcore_barrier()

        pltpu.sync_copy(rcv3.at[wid], idx)

        rbufs = [(rows, rsem), (rows2, rsem2)]
        pltpu.async_copy(e_hbm.at[pl.ds(base, CH)], rows, rsem)

        def body(j, carry):
            for m in range(2):
                @pl.when(j % 2 == m)
                def _(m=m):
                    rb, rs = rbufs[m]
                    nb, ns = rbufs[1 - m]

                    @pl.when(j + 1 < nch)
                    def _():
                        pltpu.async_copy(
                            e_hbm.at[pl.ds(base + (j + 1) * CH, CH)], nb, ns)

                    pltpu.make_async_copy(
                        e_hbm.at[pl.ds(base, CH)], rb, rs).wait()
                    pltpu.sync_copy(rb, acc.at[idx.at[j]], add=True)
            return carry

        lax.fori_loop(0, nch, body, 0)
        plsc.subcore_barrier()

        for kk in range(ncopy):
            pltpu.sync_copy(acc.at[pl.ds(row0 + kk * rc, rc)], zbuf)

            @pl.when(c == 0)
            def _():
                pltpu.sync_copy(zbuf, out0.at[pl.ds(row0 + kk * rc, rc)])

            @pl.when(c == 1)
            def _():
                pltpu.sync_copy(zbuf, out1.at[pl.ds(row0 + kk * rc, rc)])

    return k


def kernel(position_sequence, senders, receivers,
           enc_node_W1, enc_node_b1, enc_node_W2, enc_node_b2,
           enc_edge_W1, enc_edge_b1, enc_edge_W2, enc_edge_b2,
           proc_edge_W1, proc_edge_b1, proc_edge_W2, proc_edge_b2,
           proc_node_W1, proc_node_b1, proc_node_W2, proc_node_b2,
           dec_W1, dec_b1, dec_W2, dec_b2):
    n, t, d = position_sequence.shape
    e = senders.shape[0]
    s_steps = proc_edge_W1.shape[0]
    L = enc_node_W2.shape[1]
    H = enc_node_W1.shape[1]
    f32 = jnp.float32

    ps = position_sequence.astype(f32)
    vel = (ps[:, 1:] - ps[:, :-1]).reshape(n, (t - 1) * d)
    nf16 = jnp.pad(vel, ((0, 0), (0, 16 - (t - 1) * d)))
    lp = ps[:, -1]
    pp = ps[:, -2]
    lp128 = jnp.pad(lp, ((0, 0), (0, 128 - d)))
    nlp128 = -lp128
    pp128 = jnp.pad(pp, ((0, 0), (0, 128 - d)))

    snd3 = senders.astype(jnp.int32).reshape(NW, (e // NW) // CH, CH)
    rcv3 = receivers.astype(jnp.int32).reshape(NW, (e // NW) // CH, CH)

    nW1p = jnp.pad(enc_node_W1, ((0, 16 - enc_node_W1.shape[0]), (0, 0)))
    eW1p = jnp.pad(enc_edge_W1, ((0, 128 - enc_edge_W1.shape[0]), (0, 0)))
    dW2p = jnp.pad(dec_W2, ((0, 0), (0, 128 - dec_W2.shape[1])))
    db2p = jnp.pad(dec_b2, (0, 128 - dec_b2.shape[0]))

    r1 = lambda b: b.reshape(1, -1)

    peW1e = proc_edge_W1[:, :L]
    peW1s = proc_edge_W1[:, L:2 * L]
    peW1r = proc_edge_W1[:, 2 * L:]
    pnW1v = proc_node_W1[:, :L]
    pnW1a = proc_node_W1[:, L:]
    nxt = list(range(1, s_steps)) + [s_steps - 1]
    Wsn = peW1s[jnp.array(nxt)]
    Wrn = peW1r[jnp.array(nxt)]

    gather = _make_gather(n, e)
    scatter = _make_scatter(n, e)

    ngrid = n // BLKN
    egrid = e // BLKE

    v0, pvs0, pvr0 = _tc_call(
        _node_encode_body, ngrid,
        [_rows(BLKN, 16), _full((16, H)), _full((1, H)), _full((H, L)),
         _full((1, L)), _full((L, L)), _full((L, L))],
        [_rows(BLKN, L)] * 3,
        [jax.ShapeDtypeStruct((n, L), f32)] * 3,
    )(nf16, nW1p, r1(enc_node_b1), enc_node_W2, r1(enc_node_b2),
      peW1s[0], peW1r[0])

    d_rel = gather(lp128, nlp128, snd3, rcv3)

    e0 = _tc_call(
        _edge_enc_body, egrid,
        [_rows(BLKE, 128),
         _full((128, H)), _full((1, H)), _full((H, L)), _full((1, L))],
        _rows(BLKE, L),
        jax.ShapeDtypeStruct((e, L), f32),
    )(d_rel, eW1p, r1(enc_edge_b1), enc_edge_W2, r1(enc_edge_b2))

    def body(carry, ws):
        v, e_lat, pvs, pvr = carry
        (W1e, pb1, pW2, pb2, W1v, W1a, nb1, nW2, nb2, Ws_n, Wr_n) = ws
        g = gather(pvs, pvr, snd3, rcv3)
        e_lat = _tc_call(
            _edge_upd_body, egrid,
            [_rows(BLKE, L), _rows(BLKE, L),
             _full((L, H)), _full((1, H)), _full((H, L)), _full((1, L))],
            _rows(BLKE, L),
            jax.ShapeDtypeStruct((e, L), f32),
        )(e_lat, g, W1e, pb1, pW2, pb2)
        p0, p1 = scatter(e_lat, rcv3)
        v, pvs, pvr = _tc_call(
            _node_upd_proj_body, ngrid,
            [_rows(BLKN, L), _rows(BLKN, L), _rows(BLKN, L),
             _full((L, H)), _full((L, H)), _full((1, H)),
             _full((H, L)), _full((1, L)), _full((L, L)), _full((L, L))],
            [_rows(BLKN, L)] * 3,
            [jax.ShapeDtypeStruct((n, L), f32)] * 3,
        )(v, p0, p1, W1v, W1a, nb1, nW2, nb2, Ws_n, Wr_n)
        return (v, e_lat, pvs, pvr), 0

    ws_stacked = (peW1e, proc_edge_b1[:, None, :], proc_edge_W2,
                  proc_edge_b2[:, None, :], pnW1v, pnW1a,
                  proc_node_b1[:, None, :], proc_node_W2,
                  proc_node_b2[:, None, :], Wsn, Wrn)
    (v3, _, _, _), _ = lax.scan(body, (v0, e0, pvs0, pvr0), ws_stacked)

    out128 = _tc_call(
        _decode_body, ngrid,
        [_rows(BLKN, L), _full((L, H)), _full((1, H)),
         _full((H, 128)), _full((1, 128)),
         _rows(BLKN, 128), _rows(BLKN, 128)],
        _rows(BLKN, 128),
        jax.ShapeDtypeStruct((n, 128), f32),
    )(v3, dec_W1, r1(dec_b1), dW2p, r1(db2p), lp128, pp128)

    return out128[:, :d]


# speedup vs baseline: 4.1461x; 1.3589x over previous
import functools

import jax
import jax.numpy as jnp
from jax import lax
from jax.experimental import pallas as pl
from jax.experimental.pallas import tpu as pltpu
from jax.experimental.pallas import tpu_sc as plsc

NC = 2
NS = 16
NW = NC * NS

CH = 80
BLKE = 1280
BLKN = 1000
NPAD = 10240


def _ln(x):
    m = jnp.mean(x, axis=-1, keepdims=True)
    d = x - m
    v = jnp.mean(d * d, axis=-1, keepdims=True)
    return d * lax.rsqrt(v + 1e-6)


def _relu(x):
    return jnp.maximum(x, 0.0)


def _dot(a, b):
    return jnp.dot(a, b, preferred_element_type=jnp.float32)


def _node_encode_body(nf, W1, b1, W2, b2, Ws, Wr, v_o, ps_o, pr_o):
    h = _relu(_dot(nf[...], W1[...]) + b1[...])
    v = _ln(_dot(h, W2[...]) + b2[...])
    v_o[...] = v
    ps_o[...] = _dot(v, Ws[...])
    pr_o[...] = _dot(v, Wr[...])


def _edge_enc_body(d_in, eW1, eb1, eW2, eb2, e_o):
    d = d_in[...]
    dist = jnp.sqrt(jnp.sum(d * d, axis=-1, keepdims=True))
    lane = lax.broadcasted_iota(jnp.int32, d.shape, 1)
    feat = d + jnp.where(lane == 3, dist, 0.0)
    h = _relu(_dot(feat, eW1[...]) + eb1[...])
    e_o[...] = _ln(_dot(h, eW2[...]) + eb2[...])


def _edge_upd_body(e, g, W1e, b1, W2, b2, e_o):
    x = e[...]
    h = _relu(_dot(x, W1e[...]) + g[...] + b1[...])
    e_o[...] = x + _ln(_dot(h, W2[...]) + b2[...])


def _node_upd_proj_body(v, p0, p1, W1v, W1a, b1, W2, b2, Ws, Wr,
                        v_o, ps_o, pr_o):
    x = v[...]
    agg = p0[...] + p1[...]
    h = _relu(_dot(x, W1v[...]) + _dot(agg, W1a[...]) + b1[...])
    vn = x + _ln(_dot(h, W2[...]) + b2[...])
    v_o[...] = vn
    ps_o[...] = _dot(vn, Ws[...])
    pr_o[...] = _dot(vn, Wr[...])


def _decode_body(v, dW1, db1, dW2, db2, lp, pp, out):
    hd = _relu(_dot(v[...], dW1[...]) + db1[...])
    acc = _dot(hd, dW2[...]) + db2[...]
    out[...] = 2.0 * lp[...] - pp[...] + acc


def _full(shape):
    return pl.BlockSpec(shape, lambda i: (0,) * len(shape))


def _rows(blk, width):
    return pl.BlockSpec((blk, width), lambda i: (i, 0))


def _rows_off(blk, width, off_blocks):
    return pl.BlockSpec((blk, width), lambda i: (i + off_blocks, 0))


def _tc_call(body, grid, in_specs, out_specs, out_shape):
    return pl.pallas_call(
        body,
        grid=(grid,),
        in_specs=in_specs,
        out_specs=out_specs,
        out_shape=out_shape,
    )


def _sc_mesh():
    return plsc.VectorSubcoreMesh(core_axis_name="c", subcore_axis_name="s")


@functools.lru_cache(maxsize=None)
def _make_gather(n, e):
    epw = e // NW
    nch = epw // CH
    f32 = jnp.float32

    @functools.partial(
        pl.kernel,
        mesh=_sc_mesh(),
        out_type=jax.ShapeDtypeStruct((e, 128), f32),
        scratch_types=[
            pltpu.VMEM((nch, CH), jnp.int32),
            pltpu.VMEM((nch, CH), jnp.int32),
        ] + [pltpu.VMEM((CH, 128), f32)] * 4
          + [pltpu.SemaphoreType.DMA] * 8,
    )
    def k(ta, tb, snd3, rcv3, g_o, sidx, ridx,
          b0, b1, b2, b3, sa0, sa1, sa2, sa3, sb0, sb1, sb2, sb3):
        wid = lax.axis_index("s") * NC + lax.axis_index("c")
        base = wid * epw
        pltpu.sync_copy(snd3.at[wid], sidx)
        pltpu.sync_copy(rcv3.at[wid], ridx)

        bufs = [(b0, sa0, sb0), (b1, sa1, sb1), (b2, sa2, sb2),
                (b3, sa3, sb3)]

        def body(j, carry):
            @pl.when(j >= 2)
            def _():
                c3 = j - 2
                for m in range(4):
                    @pl.when(c3 % 4 == m)
                    def _(m=m):
                        b, sa, sb = bufs[m]
                        pltpu.make_async_copy(tb.at[ridx.at[0]], b, sb).wait()
                        pltpu.async_copy(
                            b, g_o.at[pl.ds(base + c3 * CH, CH)], sa)

            @pl.when(jnp.logical_and(j >= 1, j <= nch))
            def _():
                c2 = j - 1
                for m in range(4):
                    @pl.when(c2 % 4 == m)
                    def _(m=m):
                        b, sa, sb = bufs[m]
                        pltpu.make_async_copy(ta.at[sidx.at[0]], b, sa).wait()
                        pltpu.async_copy(tb.at[ridx.at[c2]], b, sb, add=True)

            @pl.when(j < nch)
            def _():
                for m in range(4):
                    @pl.when(j % 4 == m)
                    def _(m=m):
                        b, sa, sb = bufs[m]

                        @pl.when(j >= 4)
                        def _():
                            pltpu.make_async_copy(
                                b, g_o.at[pl.ds(base, CH)], sa).wait()

                        pltpu.async_copy(ta.at[sidx.at[j]], b, sa)
            return carry

        lax.fori_loop(0, nch + 2, body, 0)
        for m in range(4):
            b, sa, sb = bufs[m]
            pltpu.make_async_copy(b, g_o.at[pl.ds(base, CH)], sa).wait()

    return k


@functools.lru_cache(maxsize=None)
def _make_scatter(n, e):
    epw = e // NW
    nch = epw // CH
    rpt = NPAD // NS
    rc = 80
    ncopy = rpt // rc
    f32 = jnp.float32

    @functools.partial(
        pl.kernel,
        mesh=_sc_mesh(),
        out_type=[
            jax.ShapeDtypeStruct((NPAD, 128), f32),
            jax.ShapeDtypeStruct((NPAD, 128), f32),
        ],
        scratch_types=[
            pltpu.VMEM((nch, CH), jnp.int32),
            pltpu.VMEM((CH, 128), f32),
            pltpu.VMEM((CH, 128), f32),
            pltpu.VMEM((rc, 128), f32),
            pltpu.VMEM_SHARED((NPAD, 128), f32),
            pltpu.SemaphoreType.DMA,
            pltpu.SemaphoreType.DMA,
        ],
    )
    def k(e_hbm, rcv3, out0, out1, idx, rows, rows2, zbuf, acc, rsem, rsem2):
        c = lax.axis_index("c")
        s = lax.axis_index("s")
        wid = s * NC + c
        base = wid * epw
        row0 = s * rpt

        def zb(i, carry):
            r = i // 8
            l = (i % 8) * 16
            zbuf[r, pl.ds(l, 16)] = jnp.zeros((16,), f32)
            return carry

        lax.fori_loop(0, rc * 8, zb, 0)
        for kk in range(ncopy):
            pltpu.sync_copy(zbuf, acc.at[pl.ds(row0 + kk * rc, rc)])
        plsc.subcore_barrier()

        pltpu.sync_copy(rcv3.at[wid], idx)

        rbufs = [(rows, rsem), (rows2, rsem2)]
        pltpu.async_copy(e_hbm.at[pl.ds(base, CH)], rows, rsem)

        def body(j, carry):
            for m in range(2):
                @pl.when(j % 2 == m)
                def _(m=m):
                    rb, rs = rbufs[m]
                    nb, ns = rbufs[1 - m]

                    @pl.when(j + 1 < nch)
                    def _():
                        pltpu.async_copy(
                            e_hbm.at[pl.ds(base + (j + 1) * CH, CH)], nb, ns)

                    pltpu.make_async_copy(
                        e_hbm.at[pl.ds(base, CH)], rb, rs).wait()
                    pltpu.sync_copy(rb, acc.at[idx.at[j]], add=True)
            return carry

        lax.fori_loop(0, nch, body, 0)
        plsc.subcore_barrier()

        for kk in range(ncopy):
            pltpu.sync_copy(acc.at[pl.ds(row0 + kk * rc, rc)], zbuf)

            @pl.when(c == 0)
            def _():
                pltpu.sync_copy(zbuf, out0.at[pl.ds(row0 + kk * rc, rc)])

            @pl.when(c == 1)
            def _():
                pltpu.sync_copy(zbuf, out1.at[pl.ds(row0 + kk * rc, rc)])

    return k


def kernel(position_sequence, senders, receivers,
           enc_node_W1, enc_node_b1, enc_node_W2, enc_node_b2,
           enc_edge_W1, enc_edge_b1, enc_edge_W2, enc_edge_b2,
           proc_edge_W1, proc_edge_b1, proc_edge_W2, proc_edge_b2,
           proc_node_W1, proc_node_b1, proc_node_W2, proc_node_b2,
           dec_W1, dec_b1, dec_W2, dec_b2):
    n, t, d = position_sequence.shape
    e = senders.shape[0]
    s_steps = proc_edge_W1.shape[0]
    L = enc_node_W2.shape[1]
    H = enc_node_W1.shape[1]
    f32 = jnp.float32

    ps = position_sequence.astype(f32)
    vel = (ps[:, 1:] - ps[:, :-1]).reshape(n, (t - 1) * d)
    nf16 = jnp.pad(vel, ((0, 0), (0, 16 - (t - 1) * d)))
    lp = ps[:, -1]
    pp = ps[:, -2]
    lp128 = jnp.pad(lp, ((0, 0), (0, 128 - d)))
    nlp128 = -lp128
    pp128 = jnp.pad(pp, ((0, 0), (0, 128 - d)))

    snd3 = senders.astype(jnp.int32).reshape(NW, (e // NW) // CH, CH)
    rcv3 = receivers.astype(jnp.int32).reshape(NW, (e // NW) // CH, CH)

    nW1p = jnp.pad(enc_node_W1, ((0, 16 - enc_node_W1.shape[0]), (0, 0)))
    eW1p = jnp.pad(enc_edge_W1, ((0, 128 - enc_edge_W1.shape[0]), (0, 0)))
    dW2p = jnp.pad(dec_W2, ((0, 0), (0, 128 - dec_W2.shape[1])))
    db2p = jnp.pad(dec_b2, (0, 128 - dec_b2.shape[0]))

    r1 = lambda b: b.reshape(1, -1)

    peW1e = proc_edge_W1[:, :L]
    peW1s = proc_edge_W1[:, L:2 * L]
    peW1r = proc_edge_W1[:, 2 * L:]
    pnW1v = proc_node_W1[:, :L]
    pnW1a = proc_node_W1[:, L:]
    nxt = list(range(1, s_steps)) + [s_steps - 1]
    Wsn = peW1s[jnp.array(nxt)]
    Wrn = peW1r[jnp.array(nxt)]

    gather = _make_gather(n, e)
    scatter = _make_scatter(n, e)

    ngrid = n // BLKN
    egrid = e // BLKE

    v0, pvs0, pvr0 = _tc_call(
        _node_encode_body, ngrid,
        [_rows(BLKN, 16), _full((16, H)), _full((1, H)), _full((H, L)),
         _full((1, L)), _full((L, L)), _full((L, L))],
        [_rows(BLKN, L)] * 3,
        [jax.ShapeDtypeStruct((n, L), f32)] * 3,
    )(nf16, nW1p, r1(enc_node_b1), enc_node_W2, r1(enc_node_b2),
      peW1s[0], peW1r[0])

    d_rel = gather(lp128, nlp128, snd3, rcv3)

    e0 = _tc_call(
        _edge_enc_body, egrid,
        [_rows(BLKE, 128),
         _full((128, H)), _full((1, H)), _full((H, L)), _full((1, L))],
        _rows(BLKE, L),
        jax.ShapeDtypeStruct((e, L), f32),
    )(d_rel, eW1p, r1(enc_edge_b1), enc_edge_W2, r1(enc_edge_b2))

    def body(carry, ws):
        v, e_lat, pvs, pvr = carry
        (W1e, pb1, pW2, pb2, W1v, W1a, nb1, nW2, nb2, Ws_n, Wr_n) = ws
        g = gather(pvs, pvr, snd3, rcv3)
        e_lat = _tc_call(
            _edge_upd_body, egrid,
            [_rows(BLKE, L), _rows(BLKE, L),
             _full((L, H)), _full((1, H)), _full((H, L)), _full((1, L))],
            _rows(BLKE, L),
            jax.ShapeDtypeStruct((e, L), f32),
        )(e_lat, g, W1e, pb1, pW2, pb2)
        p0, p1 = scatter(e_lat, rcv3)
        v, pvs, pvr = _tc_call(
            _node_upd_proj_body, ngrid,
            [_rows(BLKN, L), _rows(BLKN, L), _rows(BLKN, L),
             _full((L, H)), _full((L, H)), _full((1, H)),
             _full((H, L)), _full((1, L)), _full((L, L)), _full((L, L))],
            [_rows(BLKN, L)] * 3,
            [jax.ShapeDtypeStruct((n, L), f32)] * 3,
        )(v, p0, p1, W1v, W1a, nb1, nW2, nb2, Ws_n, Wr_n)
        return (v, e_lat, pvs, pvr), 0

    ws_stacked = (peW1e, proc_edge_b1[:, None, :], proc_edge_W2,
                  proc_edge_b2[:, None, :], pnW1v, pnW1a,
                  proc_node_b1[:, None, :], proc_node_W2,
                  proc_node_b2[:, None, :], Wsn, Wrn)
    (v3, _, _, _), _ = lax.scan(body, (v0, e0, pvs0, pvr0), ws_stacked)

    out128 = _tc_call(
        _decode_body, ngrid,
        [_rows(BLKN, L), _full((L, H)), _full((1, H)),
         _full((H, 128)), _full((1, 128)),
         _rows(BLKN, 128), _rows(BLKN, 128)],
        _rows(BLKN, 128),
        jax.ShapeDtypeStruct((n, 128), f32),
    )(v3, dec_W1, r1(dec_b1), dW2p, r1(db2p), lp128, pp128)

    return out128[:, :d]


# speedup vs baseline: 4.9036x; 1.1827x over previous
import functools

import jax
import jax.numpy as jnp
from jax import lax
from jax.experimental import pallas as pl
from jax.experimental.pallas import tpu as pltpu
from jax.experimental.pallas import tpu_sc as plsc

NC = 2
NS = 16
NW = NC * NS

CH = 80
BLKE = 3200
BLKN = 2000
NPAD = 10240


def _ln(x):
    m = jnp.mean(x, axis=-1, keepdims=True)
    d = x - m
    v = jnp.mean(d * d, axis=-1, keepdims=True)
    return d * lax.rsqrt(v + 1e-6)


def _relu(x):
    return jnp.maximum(x, 0.0)


def _dot(a, b):
    return jnp.dot(a, b, preferred_element_type=jnp.float32)


def _node_encode_body(nf, W1, b1, W2, b2, Ws, Wr, v_o, ps_o, pr_o):
    h = _relu(_dot(nf[...], W1[...]) + b1[...])
    v = _ln(_dot(h, W2[...]) + b2[...])
    v_o[...] = v
    ps_o[...] = _dot(v, Ws[...])
    pr_o[...] = _dot(v, Wr[...])


def _edge_enc_body(d_in, eW1, eb1, eW2, eb2, e_o):
    d = d_in[...]
    dist = jnp.sqrt(jnp.sum(d * d, axis=-1, keepdims=True))
    lane = lax.broadcasted_iota(jnp.int32, d.shape, 1)
    feat = d + jnp.where(lane == 3, dist, 0.0)
    h = _relu(_dot(feat, eW1[...]) + eb1[...])
    e_o[...] = _ln(_dot(h, eW2[...]) + eb2[...])


def _edge_upd_body(e, g, W1e, b1, W2, b2, e_o):
    x = e[...]
    h = _relu(_dot(x, W1e[...]) + g[...] + b1[...])
    e_o[...] = x + _ln(_dot(h, W2[...]) + b2[...])


def _node_upd_proj_body(v, p0, p1, W1v, W1a, b1, W2, b2, Ws, Wr,
                        v_o, ps_o, pr_o):
    x = v[...]
    agg = p0[...] + p1[...]
    h = _relu(_dot(x, W1v[...]) + _dot(agg, W1a[...]) + b1[...])
    vn = x + _ln(_dot(h, W2[...]) + b2[...])
    v_o[...] = vn
    ps_o[...] = _dot(vn, Ws[...])
    pr_o[...] = _dot(vn, Wr[...])


def _decode_body(v, dW1, db1, dW2, db2, lp, pp, out):
    hd = _relu(_dot(v[...], dW1[...]) + db1[...])
    acc = _dot(hd, dW2[...]) + db2[...]
    out[...] = 2.0 * lp[...] - pp[...] + acc


def _full(shape):
    return pl.BlockSpec(shape, lambda i: (0,) * len(shape))


def _rows(blk, width):
    return pl.BlockSpec((blk, width), lambda i: (i, 0))


def _rows_off(blk, width, off_blocks):
    return pl.BlockSpec((blk, width), lambda i: (i + off_blocks, 0))


def _tc_call(body, grid, in_specs, out_specs, out_shape):
    return pl.pallas_call(
        body,
        grid=(grid,),
        in_specs=in_specs,
        out_specs=out_specs,
        out_shape=out_shape,
    )


def _sc_mesh():
    return plsc.VectorSubcoreMesh(core_axis_name="c", subcore_axis_name="s")


@functools.lru_cache(maxsize=None)
def _make_gather(n, e):
    epw = e // NW
    nch = epw // CH
    f32 = jnp.float32

    @functools.partial(
        pl.kernel,
        mesh=_sc_mesh(),
        out_type=jax.ShapeDtypeStruct((e, 128), f32),
        scratch_types=[
            pltpu.VMEM((nch, CH), jnp.int32),
            pltpu.VMEM((nch, CH), jnp.int32),
        ] + [pltpu.VMEM((CH, 128), f32)] * 4
          + [pltpu.SemaphoreType.DMA] * 8,
    )
    def k(ta, tb, snd3, rcv3, g_o, sidx, ridx,
          b0, b1, b2, b3, sa0, sa1, sa2, sa3, sb0, sb1, sb2, sb3):
        wid = lax.axis_index("s") * NC + lax.axis_index("c")
        base = wid * epw
        pltpu.sync_copy(snd3.at[wid], sidx)
        pltpu.sync_copy(rcv3.at[wid], ridx)

        bufs = [(b0, sa0, sb0), (b1, sa1, sb1), (b2, sa2, sb2),
                (b3, sa3, sb3)]

        def body(j, carry):
            @pl.when(j >= 2)
            def _():
                c3 = j - 2
                for m in range(4):
                    @pl.when(c3 % 4 == m)
                    def _(m=m):
                        b, sa, sb = bufs[m]
                        pltpu.make_async_copy(tb.at[ridx.at[0]], b, sb).wait()
                        pltpu.async_copy(
                            b, g_o.at[pl.ds(base + c3 * CH, CH)], sa)

            @pl.when(jnp.logical_and(j >= 1, j <= nch))
            def _():
                c2 = j - 1
                for m in range(4):
                    @pl.when(c2 % 4 == m)
                    def _(m=m):
                        b, sa, sb = bufs[m]
                        pltpu.make_async_copy(ta.at[sidx.at[0]], b, sa).wait()
                        pltpu.async_copy(tb.at[ridx.at[c2]], b, sb, add=True)

            @pl.when(j < nch)
            def _():
                for m in range(4):
                    @pl.when(j % 4 == m)
                    def _(m=m):
                        b, sa, sb = bufs[m]

                        @pl.when(j >= 4)
                        def _():
                            pltpu.make_async_copy(
                                b, g_o.at[pl.ds(base, CH)], sa).wait()

                        pltpu.async_copy(ta.at[sidx.at[j]], b, sa)
            return carry

        lax.fori_loop(0, nch + 2, body, 0)
        for m in range(4):
            b, sa, sb = bufs[m]
            pltpu.make_async_copy(b, g_o.at[pl.ds(base, CH)], sa).wait()

    return k


@functools.lru_cache(maxsize=None)
def _make_scatter(n, e):
    epw = e // NW
    nch = epw // CH
    rpt = NPAD // NS
    rc = 80
    ncopy = rpt // rc
    f32 = jnp.float32

    @functools.partial(
        pl.kernel,
        mesh=_sc_mesh(),
        out_type=[
            jax.ShapeDtypeStruct((NPAD, 128), f32),
            jax.ShapeDtypeStruct((NPAD, 128), f32),
        ],
        scratch_types=[
            pltpu.VMEM((nch, CH), jnp.int32),
            pltpu.VMEM((CH, 128), f32),
            pltpu.VMEM((CH, 128), f32),
            pltpu.VMEM((rc, 128), f32),
            pltpu.VMEM_SHARED((NPAD, 128), f32),
            pltpu.SemaphoreType.DMA,
            pltpu.SemaphoreType.DMA,
        ],
    )
    def k(e_hbm, rcv3, out0, out1, idx, rows, rows2, zbuf, acc, rsem, rsem2):
        c = lax.axis_index("c")
        s = lax.axis_index("s")
        wid = s * NC + c
        base = wid * epw
        row0 = s * rpt

        def zb(i, carry):
            r = i // 8
            l = (i % 8) * 16
            zbuf[r, pl.ds(l, 16)] = jnp.zeros((16,), f32)
            return carry

        lax.fori_loop(0, rc * 8, zb, 0)
        for kk in range(ncopy):
            pltpu.sync_copy(zbuf, acc.at[pl.ds(row0 + kk * rc, rc)])
        plsc.subcore_barrier()

        pltpu.sync_copy(rcv3.at[wid], idx)

        rbufs = [(rows, rsem), (rows2, rsem2)]
        pltpu.async_copy(e_hbm.at[pl.ds(base, CH)], rows, rsem)

        def body(j, carry):
            for m in range(2):
                @pl.when(j % 2 == m)
                def _(m=m):
                    rb, rs = rbufs[m]
                    nb, ns = rbufs[1 - m]

                    @pl.when(j + 1 < nch)
                    def _():
                        pltpu.async_copy(
                            e_hbm.at[pl.ds(base + (j + 1) * CH, CH)], nb, ns)

                    pltpu.make_async_copy(
                        e_hbm.at[pl.ds(base, CH)], rb, rs).wait()
                    pltpu.sync_copy(rb, acc.at[idx.at[j]], add=True)
            return carry

        lax.fori_loop(0, nch, body, 0)
        plsc.subcore_barrier()

        for kk in range(ncopy):
            pltpu.sync_copy(acc.at[pl.ds(row0 + kk * rc, rc)], zbuf)

            @pl.when(c == 0)
            def _():
                pltpu.sync_copy(zbuf, out0.at[pl.ds(row0 + kk * rc, rc)])

            @pl.when(c == 1)
            def _():
                pltpu.sync_copy(zbuf, out1.at[pl.ds(row0 + kk * rc, rc)])

    return k


def kernel(position_sequence, senders, receivers,
           enc_node_W1, enc_node_b1, enc_node_W2, enc_node_b2,
           enc_edge_W1, enc_edge_b1, enc_edge_W2, enc_edge_b2,
           proc_edge_W1, proc_edge_b1, proc_edge_W2, proc_edge_b2,
           proc_node_W1, proc_node_b1, proc_node_W2, proc_node_b2,
           dec_W1, dec_b1, dec_W2, dec_b2):
    n, t, d = position_sequence.shape
    e = senders.shape[0]
    s_steps = proc_edge_W1.shape[0]
    L = enc_node_W2.shape[1]
    H = enc_node_W1.shape[1]
    f32 = jnp.float32

    ps = position_sequence.astype(f32)
    vel = (ps[:, 1:] - ps[:, :-1]).reshape(n, (t - 1) * d)
    nf16 = jnp.pad(vel, ((0, 0), (0, 16 - (t - 1) * d)))
    lp = ps[:, -1]
    pp = ps[:, -2]
    lp128 = jnp.pad(lp, ((0, 0), (0, 128 - d)))
    nlp128 = -lp128
    pp128 = jnp.pad(pp, ((0, 0), (0, 128 - d)))

    snd3 = senders.astype(jnp.int32).reshape(NW, (e // NW) // CH, CH)
    rcv3 = receivers.astype(jnp.int32).reshape(NW, (e // NW) // CH, CH)

    nW1p = jnp.pad(enc_node_W1, ((0, 16 - enc_node_W1.shape[0]), (0, 0)))
    eW1p = jnp.pad(enc_edge_W1, ((0, 128 - enc_edge_W1.shape[0]), (0, 0)))
    dW2p = jnp.pad(dec_W2, ((0, 0), (0, 128 - dec_W2.shape[1])))
    db2p = jnp.pad(dec_b2, (0, 128 - dec_b2.shape[0]))

    r1 = lambda b: b.reshape(1, -1)

    peW1e = proc_edge_W1[:, :L]
    peW1s = proc_edge_W1[:, L:2 * L]
    peW1r = proc_edge_W1[:, 2 * L:]
    pnW1v = proc_node_W1[:, :L]
    pnW1a = proc_node_W1[:, L:]
    nxt = list(range(1, s_steps)) + [s_steps - 1]
    Wsn = peW1s[jnp.array(nxt)]
    Wrn = peW1r[jnp.array(nxt)]

    gather = _make_gather(n, e)
    scatter = _make_scatter(n, e)

    ngrid = n // BLKN
    egrid = e // BLKE

    v0, pvs0, pvr0 = _tc_call(
        _node_encode_body, ngrid,
        [_rows(BLKN, 16), _full((16, H)), _full((1, H)), _full((H, L)),
         _full((1, L)), _full((L, L)), _full((L, L))],
        [_rows(BLKN, L)] * 3,
        [jax.ShapeDtypeStruct((n, L), f32)] * 3,
    )(nf16, nW1p, r1(enc_node_b1), enc_node_W2, r1(enc_node_b2),
      peW1s[0], peW1r[0])

    d_rel = gather(lp128, nlp128, snd3, rcv3)

    e0 = _tc_call(
        _edge_enc_body, egrid,
        [_rows(BLKE, 128),
         _full((128, H)), _full((1, H)), _full((H, L)), _full((1, L))],
        _rows(BLKE, L),
        jax.ShapeDtypeStruct((e, L), f32),
    )(d_rel, eW1p, r1(enc_edge_b1), enc_edge_W2, r1(enc_edge_b2))

    def body(carry, ws):
        v, e_lat, pvs, pvr = carry
        (W1e, pb1, pW2, pb2, W1v, W1a, nb1, nW2, nb2, Ws_n, Wr_n) = ws
        g = gather(pvs, pvr, snd3, rcv3)
        e_lat = _tc_call(
            _edge_upd_body, egrid,
            [_rows(BLKE, L), _rows(BLKE, L),
             _full((L, H)), _full((1, H)), _full((H, L)), _full((1, L))],
            _rows(BLKE, L),
            jax.ShapeDtypeStruct((e, L), f32),
        )(e_lat, g, W1e, pb1, pW2, pb2)
        p0, p1 = scatter(e_lat, rcv3)
        v, pvs, pvr = _tc_call(
            _node_upd_proj_body, ngrid,
            [_rows(BLKN, L), _rows(BLKN, L), _rows(BLKN, L),
             _full((L, H)), _full((L, H)), _full((1, H)),
             _full((H, L)), _full((1, L)), _full((L, L)), _full((L, L))],
            [_rows(BLKN, L)] * 3,
            [jax.ShapeDtypeStruct((n, L), f32)] * 3,
        )(v, p0, p1, W1v, W1a, nb1, nW2, nb2, Ws_n, Wr_n)
        return (v, e_lat, pvs, pvr), 0

    ws_stacked = (peW1e, proc_edge_b1[:, None, :], proc_edge_W2,
                  proc_edge_b2[:, None, :], pnW1v, pnW1a,
                  proc_node_b1[:, None, :], proc_node_W2,
                  proc_node_b2[:, None, :], Wsn, Wrn)
    (v3, _, _, _), _ = lax.scan(body, (v0, e0, pvs0, pvr0), ws_stacked)

    out128 = _tc_call(
        _decode_body, ngrid,
        [_rows(BLKN, L), _full((L, H)), _full((1, H)),
         _full((H, 128)), _full((1, 128)),
         _rows(BLKN, 128), _rows(BLKN, 128)],
        _rows(BLKN, 128),
        jax.ShapeDtypeStruct((n, 128), f32),
    )(v3, dec_W1, r1(dec_b1), dW2p, r1(db2p), lp128, pp128)

    return out128[:, :d]


# speedup vs baseline: 5.2064x; 1.0617x over previous
import functools

import jax
import jax.numpy as jnp
from jax import lax
from jax.experimental import pallas as pl
from jax.experimental.pallas import tpu as pltpu
from jax.experimental.pallas import tpu_sc as plsc

NC = 2
NS = 16
NW = NC * NS

CH = 80
BLKE = 6400
BLKN = 2000
NPAD = 10240


def _ln(x):
    m = jnp.mean(x, axis=-1, keepdims=True)
    d = x - m
    v = jnp.mean(d * d, axis=-1, keepdims=True)
    return d * lax.rsqrt(v + 1e-6)


def _relu(x):
    return jnp.maximum(x, 0.0)


def _dot(a, b):
    return jnp.dot(a, b, preferred_element_type=jnp.float32)


def _node_encode_body(nf, W1, b1, W2, b2, Ws, Wr, v_o, ps_o, pr_o):
    h = _relu(_dot(nf[...], W1[...]) + b1[...])
    v = _ln(_dot(h, W2[...]) + b2[...])
    v_o[...] = v
    ps_o[...] = _dot(v, Ws[...])
    pr_o[...] = _dot(v, Wr[...])


def _edge_enc_body(d_in, eW1, eb1, eW2, eb2, e_o):
    d = d_in[...]
    dist = jnp.sqrt(jnp.sum(d * d, axis=-1, keepdims=True))
    lane = lax.broadcasted_iota(jnp.int32, d.shape, 1)
    feat = d + jnp.where(lane == 3, dist, 0.0)
    h = _relu(_dot(feat, eW1[...]) + eb1[...])
    e_o[...] = _ln(_dot(h, eW2[...]) + eb2[...])


def _edge_upd_body(e, g, W1e, b1, W2, b2, e_o):
    x = e[...]
    h = _relu(_dot(x, W1e[...]) + g[...] + b1[...])
    e_o[...] = x + _ln(_dot(h, W2[...]) + b2[...])


def _node_upd_proj_body(v, p0, p1, W1v, W1a, b1, W2, b2, Ws, Wr,
                        v_o, ps_o, pr_o):
    x = v[...]
    agg = p0[...] + p1[...]
    h = _relu(_dot(x, W1v[...]) + _dot(agg, W1a[...]) + b1[...])
    vn = x + _ln(_dot(h, W2[...]) + b2[...])
    v_o[...] = vn
    ps_o[...] = _dot(vn, Ws[...])
    pr_o[...] = _dot(vn, Wr[...])


def _decode_body(v, dW1, db1, dW2, db2, lp, pp, out):
    hd = _relu(_dot(v[...], dW1[...]) + db1[...])
    acc = _dot(hd, dW2[...]) + db2[...]
    out[...] = 2.0 * lp[...] - pp[...] + acc


def _full(shape):
    return pl.BlockSpec(shape, lambda i: (0,) * len(shape))


def _rows(blk, width):
    return pl.BlockSpec((blk, width), lambda i: (i, 0))


def _rows_off(blk, width, off_blocks):
    return pl.BlockSpec((blk, width), lambda i: (i + off_blocks, 0))


def _tc_call(body, grid, in_specs, out_specs, out_shape):
    return pl.pallas_call(
        body,
        grid=(grid,),
        in_specs=in_specs,
        out_specs=out_specs,
        out_shape=out_shape,
    )


def _sc_mesh():
    return plsc.VectorSubcoreMesh(core_axis_name="c", subcore_axis_name="s")


@functools.lru_cache(maxsize=None)
def _make_gather(n, e):
    epw = e // NW
    nch = epw // CH
    f32 = jnp.float32

    @functools.partial(
        pl.kernel,
        mesh=_sc_mesh(),
        out_type=jax.ShapeDtypeStruct((e, 128), f32),
        scratch_types=[
            pltpu.VMEM((nch, CH), jnp.int32),
            pltpu.VMEM((nch, CH), jnp.int32),
        ] + [pltpu.VMEM((CH, 128), f32)] * 4
          + [pltpu.SemaphoreType.DMA] * 8,
    )
    def k(ta, tb, snd3, rcv3, g_o, sidx, ridx,
          b0, b1, b2, b3, sa0, sa1, sa2, sa3, sb0, sb1, sb2, sb3):
        wid = lax.axis_index("s") * NC + lax.axis_index("c")
        base = wid * epw
        pltpu.sync_copy(snd3.at[wid], sidx)
        pltpu.sync_copy(rcv3.at[wid], ridx)

        bufs = [(b0, sa0, sb0), (b1, sa1, sb1), (b2, sa2, sb2),
                (b3, sa3, sb3)]

        def body(j, carry):
            @pl.when(j >= 2)
            def _():
                c3 = j - 2
                for m in range(4):
                    @pl.when(c3 % 4 == m)
                    def _(m=m):
                        b, sa, sb = bufs[m]
                        pltpu.make_async_copy(tb.at[ridx.at[0]], b, sb).wait()
                        pltpu.async_copy(
                            b, g_o.at[pl.ds(base + c3 * CH, CH)], sa)

            @pl.when(jnp.logical_and(j >= 1, j <= nch))
            def _():
                c2 = j - 1
                for m in range(4):
                    @pl.when(c2 % 4 == m)
                    def _(m=m):
                        b, sa, sb = bufs[m]
                        pltpu.make_async_copy(ta.at[sidx.at[0]], b, sa).wait()
                        pltpu.async_copy(tb.at[ridx.at[c2]], b, sb, add=True)

            @pl.when(j < nch)
            def _():
                for m in range(4):
                    @pl.when(j % 4 == m)
                    def _(m=m):
                        b, sa, sb = bufs[m]

                        @pl.when(j >= 4)
                        def _():
                            pltpu.make_async_copy(
                                b, g_o.at[pl.ds(base, CH)], sa).wait()

                        pltpu.async_copy(ta.at[sidx.at[j]], b, sa)
            return carry

        lax.fori_loop(0, nch + 2, body, 0)
        for m in range(4):
            b, sa, sb = bufs[m]
            pltpu.make_async_copy(b, g_o.at[pl.ds(base, CH)], sa).wait()

    return k


@functools.lru_cache(maxsize=None)
def _make_scatter(n, e):
    epw = e // NW
    nch = epw // CH
    rpt = NPAD // NS
    rc = 80
    ncopy = rpt // rc
    f32 = jnp.float32

    @functools.partial(
        pl.kernel,
        mesh=_sc_mesh(),
        out_type=[
            jax.ShapeDtypeStruct((NPAD, 128), f32),
            jax.ShapeDtypeStruct((NPAD, 128), f32),
        ],
        scratch_types=[
            pltpu.VMEM((nch, CH), jnp.int32),
            pltpu.VMEM((CH, 128), f32),
            pltpu.VMEM((CH, 128), f32),
            pltpu.VMEM((rc, 128), f32),
            pltpu.VMEM_SHARED((NPAD, 128), f32),
            pltpu.SemaphoreType.DMA,
            pltpu.SemaphoreType.DMA,
        ],
    )
    def k(e_hbm, rcv3, out0, out1, idx, rows, rows2, zbuf, acc, rsem, rsem2):
        c = lax.axis_index("c")
        s = lax.axis_index("s")
        wid = s * NC + c
        base = wid * epw
        row0 = s * rpt

        def zb(i, carry):
            r = i // 8
            l = (i % 8) * 16
            zbuf[r, pl.ds(l, 16)] = jnp.zeros((16,), f32)
            return carry

        lax.fori_loop(0, rc * 8, zb, 0)
        for kk in range(ncopy):
            pltpu.sync_copy(zbuf, acc.at[pl.ds(row0 + kk * rc, rc)])
        plsc.subcore_barrier()

        pltpu.sync_copy(rcv3.at[wid], idx)

        rbufs = [(rows, rsem), (rows2, rsem2)]
        pltpu.async_copy(e_hbm.at[pl.ds(base, CH)], rows, rsem)

        def body(j, carry):
            for m in range(2):
                @pl.when(j % 2 == m)
                def _(m=m):
                    rb, rs = rbufs[m]
                    nb, ns = rbufs[1 - m]

                    @pl.when(j + 1 < nch)
                    def _():
                        pltpu.async_copy(
                            e_hbm.at[pl.ds(base + (j + 1) * CH, CH)], nb, ns)

                    pltpu.make_async_copy(
                        e_hbm.at[pl.ds(base, CH)], rb, rs).wait()
                    pltpu.sync_copy(rb, acc.at[idx.at[j]], add=True)
            return carry

        lax.fori_loop(0, nch, body, 0)
        plsc.subcore_barrier()

        for kk in range(ncopy):
            pltpu.sync_copy(acc.at[pl.ds(row0 + kk * rc, rc)], zbuf)

            @pl.when(c == 0)
            def _():
                pltpu.sync_copy(zbuf, out0.at[pl.ds(row0 + kk * rc, rc)])

            @pl.when(c == 1)
            def _():
                pltpu.sync_copy(zbuf, out1.at[pl.ds(row0 + kk * rc, rc)])

    return k


def kernel(position_sequence, senders, receivers,
           enc_node_W1, enc_node_b1, enc_node_W2, enc_node_b2,
           enc_edge_W1, enc_edge_b1, enc_edge_W2, enc_edge_b2,
           proc_edge_W1, proc_edge_b1, proc_edge_W2, proc_edge_b2,
           proc_node_W1, proc_node_b1, proc_node_W2, proc_node_b2,
           dec_W1, dec_b1, dec_W2, dec_b2):
    n, t, d = position_sequence.shape
    e = senders.shape[0]
    s_steps = proc_edge_W1.shape[0]
    L = enc_node_W2.shape[1]
    H = enc_node_W1.shape[1]
    f32 = jnp.float32

    ps = position_sequence.astype(f32)
    vel = (ps[:, 1:] - ps[:, :-1]).reshape(n, (t - 1) * d)
    nf16 = jnp.pad(vel, ((0, 0), (0, 16 - (t - 1) * d)))
    lp = ps[:, -1]
    pp = ps[:, -2]
    lp128 = jnp.pad(lp, ((0, 0), (0, 128 - d)))
    nlp128 = -lp128
    pp128 = jnp.pad(pp, ((0, 0), (0, 128 - d)))

    snd3 = senders.astype(jnp.int32).reshape(NW, (e // NW) // CH, CH)
    rcv3 = receivers.astype(jnp.int32).reshape(NW, (e // NW) // CH, CH)

    nW1p = jnp.pad(enc_node_W1, ((0, 16 - enc_node_W1.shape[0]), (0, 0)))
    eW1p = jnp.pad(enc_edge_W1, ((0, 128 - enc_edge_W1.shape[0]), (0, 0)))
    dW2p = jnp.pad(dec_W2, ((0, 0), (0, 128 - dec_W2.shape[1])))
    db2p = jnp.pad(dec_b2, (0, 128 - dec_b2.shape[0]))

    r1 = lambda b: b.reshape(1, -1)

    peW1e = proc_edge_W1[:, :L]
    peW1s = proc_edge_W1[:, L:2 * L]
    peW1r = proc_edge_W1[:, 2 * L:]
    pnW1v = proc_node_W1[:, :L]
    pnW1a = proc_node_W1[:, L:]
    nxt = list(range(1, s_steps)) + [s_steps - 1]
    Wsn = peW1s[jnp.array(nxt)]
    Wrn = peW1r[jnp.array(nxt)]

    gather = _make_gather(n, e)
    scatter = _make_scatter(n, e)

    ngrid = n // BLKN
    egrid = e // BLKE

    v0, pvs0, pvr0 = _tc_call(
        _node_encode_body, ngrid,
        [_rows(BLKN, 16), _full((16, H)), _full((1, H)), _full((H, L)),
         _full((1, L)), _full((L, L)), _full((L, L))],
        [_rows(BLKN, L)] * 3,
        [jax.ShapeDtypeStruct((n, L), f32)] * 3,
    )(nf16, nW1p, r1(enc_node_b1), enc_node_W2, r1(enc_node_b2),
      peW1s[0], peW1r[0])

    d_rel = gather(lp128, nlp128, snd3, rcv3)

    e0 = _tc_call(
        _edge_enc_body, egrid,
        [_rows(BLKE, 128),
         _full((128, H)), _full((1, H)), _full((H, L)), _full((1, L))],
        _rows(BLKE, L),
        jax.ShapeDtypeStruct((e, L), f32),
    )(d_rel, eW1p, r1(enc_edge_b1), enc_edge_W2, r1(enc_edge_b2))

    def body(carry, ws):
        v, e_lat, pvs, pvr = carry
        (W1e, pb1, pW2, pb2, W1v, W1a, nb1, nW2, nb2, Ws_n, Wr_n) = ws
        g = gather(pvs, pvr, snd3, rcv3)
        e_lat = _tc_call(
            _edge_upd_body, egrid,
            [_rows(BLKE, L), _rows(BLKE, L),
             _full((L, H)), _full((1, H)), _full((H, L)), _full((1, L))],
            _rows(BLKE, L),
            jax.ShapeDtypeStruct((e, L), f32),
        )(e_lat, g, W1e, pb1, pW2, pb2)
        p0, p1 = scatter(e_lat, rcv3)
        v, pvs, pvr = _tc_call(
            _node_upd_proj_body, ngrid,
            [_rows(BLKN, L), _rows(BLKN, L), _rows(BLKN, L),
             _full((L, H)), _full((L, H)), _full((1, H)),
             _full((H, L)), _full((1, L)), _full((L, L)), _full((L, L))],
            [_rows(BLKN, L)] * 3,
            [jax.ShapeDtypeStruct((n, L), f32)] * 3,
        )(v, p0, p1, W1v, W1a, nb1, nW2, nb2, Ws_n, Wr_n)
        return (v, e_lat, pvs, pvr), 0

    ws_stacked = (peW1e, proc_edge_b1[:, None, :], proc_edge_W2,
                  proc_edge_b2[:, None, :], pnW1v, pnW1a,
                  proc_node_b1[:, None, :], proc_node_W2,
                  proc_node_b2[:, None, :], Wsn, Wrn)
    (v3, _, _, _), _ = lax.scan(body, (v0, e0, pvs0, pvr0), ws_stacked)

    out128 = _tc_call(
        _decode_body, ngrid,
        [_rows(BLKN, L), _full((L, H)), _full((1, H)),
         _full((H, 128)), _full((1, 128)),
         _rows(BLKN, 128), _rows(BLKN, 128)],
        _rows(BLKN, 128),
        jax.ShapeDtypeStruct((n, 128), f32),
    )(v3, dec_W1, r1(dec_b1), dW2p, r1(db2p), lp128, pp128)

    return out128[:, :d]
